# baseline (speedup 1.0000x reference)
import functools
import jax
import jax.numpy as jnp
from jax import lax
from jax.experimental import pallas as pl
from jax.experimental.pallas import tpu as pltpu
from jax.experimental.pallas import tpu_sc as plsc

B, N, D_MODEL = 4, 8192, 1024
EMBED_DIM = 1536
N_REGIONS = 2048

R_BLK = 512
P_BLK = 2048
NW = 32
B_PER_W = N // NW
NBUF = 3
ROWS_SUB = 32
SUB = B_PER_W // ROWS_SUB


def _cond_kernel(e_ref, w_ref, r_ref, o_ref):
    o_ref[...] = jax.lax.dot_general(
        e_ref[...].astype(jnp.bfloat16), w_ref[...].astype(jnp.bfloat16),
        dimension_numbers=(((1,), (1,)), ((), ())),
        preferred_element_type=jnp.float32,
    ) + r_ref[...]


def _sc_gather_kernel(table_hbm, idx_hbm, out_hbm, idx_v, rows, gsems, ssems):
    wid = lax.axis_index("s") * 2 + lax.axis_index("c")
    base = wid * B_PER_W
    pltpu.sync_copy(idx_hbm.at[pl.ds(base, B_PER_W)], idx_v)

    gathers = {}
    stores = {}
    for c in range(NBUF):
        gathers[c] = pltpu.async_copy(
            table_hbm.at[idx_v.at[pl.ds(c * ROWS_SUB, ROWS_SUB)]],
            rows.at[c], gsems.at[c])
    for c in range(SUB):
        gathers[c].wait()
        stores[c] = pltpu.async_copy(
            rows.at[c % NBUF],
            out_hbm.at[pl.ds(base + c * ROWS_SUB, ROWS_SUB)],
            ssems.at[c % NBUF])
        nxt = c + NBUF
        if nxt < SUB:
            stores[c].wait()
            gathers[nxt] = pltpu.async_copy(
                table_hbm.at[idx_v.at[pl.ds(nxt * ROWS_SUB, ROWS_SUB)]],
                rows.at[nxt % NBUF], gsems.at[nxt % NBUF])
        else:
            stores[c].wait()


def _add_kernel(canvas_ref, cond_ref, out_ref):
    out_ref[...] = canvas_ref[...] + cond_ref[...][None]


def kernel(canvas, region_ids, embeddings, W, residuals):
    table = pl.pallas_call(
        _cond_kernel,
        grid=(N_REGIONS // R_BLK,),
        in_specs=[
            pl.BlockSpec((R_BLK, EMBED_DIM), lambda i: (i, 0)),
            pl.BlockSpec((D_MODEL, EMBED_DIM), lambda i: (0, 0)),
            pl.BlockSpec((R_BLK, D_MODEL), lambda i: (i, 0)),
        ],
        out_specs=pl.BlockSpec((R_BLK, D_MODEL), lambda i: (i, 0)),
        out_shape=jax.ShapeDtypeStruct((N_REGIONS, D_MODEL), jnp.float32),
        compiler_params=pltpu.CompilerParams(
            dimension_semantics=("parallel",)),
    )(embeddings, W, residuals)

    ids32 = region_ids.astype(jnp.int32)

    sc_gather = functools.partial(
        pl.kernel,
        mesh=plsc.VectorSubcoreMesh(core_axis_name="c", subcore_axis_name="s"),
        out_type=jax.ShapeDtypeStruct((N, D_MODEL), jnp.float32),
        scratch_types=[
            pltpu.VMEM((B_PER_W,), jnp.int32),
            pltpu.VMEM((NBUF, ROWS_SUB, D_MODEL), jnp.float32),
            pltpu.SemaphoreType.DMA((NBUF,)),
            pltpu.SemaphoreType.DMA((NBUF,)),
        ],
    )(_sc_gather_kernel)

    cond_per_pos = sc_gather(table, ids32)

    out = pl.pallas_call(
        _add_kernel,
        grid=(N // P_BLK, B),
        in_specs=[
            pl.BlockSpec((1, P_BLK, D_MODEL), lambda i, b: (b, i, 0)),
            pl.BlockSpec((P_BLK, D_MODEL), lambda i, b: (i, 0)),
        ],
        out_specs=pl.BlockSpec((1, P_BLK, D_MODEL), lambda i, b: (b, i, 0)),
        out_shape=jax.ShapeDtypeStruct((B, N, D_MODEL), jnp.float32),
        compiler_params=pltpu.CompilerParams(
            dimension_semantics=("parallel", "parallel")),
    )(canvas, cond_per_pos)

    return out
